# baseline (speedup 1.0000x reference)
import jax
import jax.numpy as jnp
from jax import lax
from jax.experimental import pallas as pl
from jax.experimental.pallas import tpu as pltpu
from jax.experimental.pallas import tpu_sc as plsc

B = 4096
L = 200
DIM = 32
LP = 256
NW = 32
RB = B // NW
C0 = 128
C1 = L - C0
SR = B * DIM // 128


def _sc_pool_body(ids_hbm, table_hbm, out_hbm, ids_v, buf0, buf1, sums_v,
                  sem0, sem1):
    wid = lax.axis_index("s") * 2 + lax.axis_index("c")
    base = wid * RB

    pltpu.sync_copy(ids_hbm.at[pl.ds(base * LP, RB * LP)], ids_v)

    def row_body(lr, carry):
        rb = lr // 8
        sl = lr - rb * 8
        off0 = rb * 2048 + sl * 128
        g0 = pltpu.async_copy(table_hbm.at[ids_v.at[pl.ds(off0, C0)]],
                              buf0, sem0)
        g1 = pltpu.async_copy(table_hbm.at[ids_v.at[pl.ds(off0 + 1024, C1)]],
                              buf1, sem1)
        g0.wait()
        g1.wait()

        a0 = jnp.zeros((16,), jnp.float32)
        a1 = jnp.zeros((16,), jnp.float32)
        a2 = jnp.zeros((16,), jnp.float32)
        a3 = jnp.zeros((16,), jnp.float32)
        for r in range(0, C0, 2):
            a0 = a0 + buf0[r, pl.ds(0, 16)]
            a1 = a1 + buf0[r, pl.ds(16, 16)]
            a2 = a2 + buf0[r + 1, pl.ds(0, 16)]
            a3 = a3 + buf0[r + 1, pl.ds(16, 16)]
        for r in range(0, C1, 2):
            a0 = a0 + buf1[r, pl.ds(0, 16)]
            a1 = a1 + buf1[r, pl.ds(16, 16)]
            a2 = a2 + buf1[r + 1, pl.ds(0, 16)]
            a3 = a3 + buf1[r + 1, pl.ds(16, 16)]

        sums_v[lr, pl.ds(0, 16)] = a0 + a2
        sums_v[lr, pl.ds(16, 16)] = a1 + a3
        return carry

    lax.fori_loop(0, RB, row_body, 0)

    pltpu.sync_copy(sums_v, out_hbm.at[pl.ds(base, RB)])


@jax.jit
def _sc_pool(ids_tiled, table):
    mesh = plsc.VectorSubcoreMesh(core_axis_name="c", subcore_axis_name="s")
    return pl.kernel(
        _sc_pool_body,
        out_type=jax.ShapeDtypeStruct((B, DIM), jnp.float32),
        mesh=mesh,
        compiler_params=pltpu.CompilerParams(use_tc_tiling_on_sc=False),
        scratch_types=[
            pltpu.VMEM((RB * LP,), jnp.int32),
            pltpu.VMEM((C0, DIM), jnp.float32),
            pltpu.VMEM((C1, DIM), jnp.float32),
            pltpu.VMEM((RB, DIM), jnp.float32),
            pltpu.SemaphoreType.DMA,
            pltpu.SemaphoreType.DMA,
        ],
    )(ids_tiled, table)


def _tc_body(ids_ref, s_ref, w_ref, b_ref, o_ref):
    cnt = jnp.sum((ids_ref[...] != 0).astype(jnp.float32), axis=1,
                  keepdims=True)
    pooled = s_ref[...] * (1.0 / jnp.maximum(cnt, 1.0))
    o_ref[...] = (
        jnp.dot(pooled, w_ref[...].T, preferred_element_type=jnp.float32)
        + b_ref[...]
    )


@jax.jit
def _tc_head(ids, sums, W, b):
    return pl.pallas_call(
        _tc_body,
        out_shape=jax.ShapeDtypeStruct((B, W.shape[0]), jnp.float32),
    )(ids, sums, W, b.reshape(1, -1))


def kernel(ids, table, W, b):
    ids32 = ids.astype(jnp.int32)
    ids_tiled = (
        jnp.pad(ids32, ((0, 0), (0, LP - L)))
        .reshape(B // 8, 8, 2, 128)
        .transpose(0, 2, 1, 3)
        .reshape(-1)
    )
    sums = _sc_pool(ids_tiled, table)
    return _tc_head(ids32, sums, W, b)


# speedup vs baseline: 1.1427x; 1.1427x over previous
import jax
import jax.numpy as jnp
from jax import lax
from jax.experimental import pallas as pl
from jax.experimental.pallas import tpu as pltpu
from jax.experimental.pallas import tpu_sc as plsc

B = 4096
L = 200
DIM = 32
LP = 256
NW = 32
RB = B // NW
C0 = 128
C1 = L - C0
SR = B * DIM // 128


def _sc_pool_body(ids_hbm, table_hbm, out_hbm, ids_v,
                  buf00, buf01, buf10, buf11, sums_v,
                  sem00, sem01, sem10, sem11):
    wid = lax.axis_index("s") * 2 + lax.axis_index("c")
    base = wid * RB

    pltpu.sync_copy(ids_hbm.at[pl.ds(base * LP, RB * LP)], ids_v)

    sets = ((buf00, buf01, sem00, sem01), (buf10, buf11, sem10, sem11))

    def issue(r, bset):
        b0, b1, s0, s1 = bset
        rb = r // 8
        sl = r - rb * 8
        off0 = rb * 2048 + sl * 128
        pltpu.async_copy(table_hbm.at[ids_v.at[pl.ds(off0, C0)]], b0, s0)
        pltpu.async_copy(table_hbm.at[ids_v.at[pl.ds(off0 + 1024, C1)]],
                         b1, s1)

    def wait(bset):
        b0, b1, s0, s1 = bset
        pltpu.make_async_copy(table_hbm.at[ids_v.at[pl.ds(0, C0)]],
                              b0, s0).wait()
        pltpu.make_async_copy(table_hbm.at[ids_v.at[pl.ds(0, C1)]],
                              b1, s1).wait()

    def consume(lr, bset):
        b0, b1, _, _ = bset
        a0 = jnp.zeros((16,), jnp.float32)
        a1 = jnp.zeros((16,), jnp.float32)
        a2 = jnp.zeros((16,), jnp.float32)
        a3 = jnp.zeros((16,), jnp.float32)
        for r in range(0, C0, 2):
            a0 = a0 + b0[r, pl.ds(0, 16)]
            a1 = a1 + b0[r, pl.ds(16, 16)]
            a2 = a2 + b0[r + 1, pl.ds(0, 16)]
            a3 = a3 + b0[r + 1, pl.ds(16, 16)]
        for r in range(0, C1, 2):
            a0 = a0 + b1[r, pl.ds(0, 16)]
            a1 = a1 + b1[r, pl.ds(16, 16)]
            a2 = a2 + b1[r + 1, pl.ds(0, 16)]
            a3 = a3 + b1[r + 1, pl.ds(16, 16)]
        sums_v[lr, pl.ds(0, 16)] = a0 + a2
        sums_v[lr, pl.ds(16, 16)] = a1 + a3

    issue(0, sets[0])
    issue(1, sets[1])

    def pair_body(i, carry):
        r = i * 2
        wait(sets[0])
        consume(r, sets[0])
        issue(jnp.minimum(r + 2, RB - 1), sets[0])
        wait(sets[1])
        consume(r + 1, sets[1])
        issue(jnp.minimum(r + 3, RB - 1), sets[1])
        return carry

    lax.fori_loop(0, RB // 2, pair_body, 0)
    wait(sets[0])
    wait(sets[1])

    pltpu.sync_copy(sums_v, out_hbm.at[pl.ds(base, RB)])


@jax.jit
def _sc_pool(ids_tiled, table):
    mesh = plsc.VectorSubcoreMesh(core_axis_name="c", subcore_axis_name="s")
    return pl.kernel(
        _sc_pool_body,
        out_type=jax.ShapeDtypeStruct((B, DIM), jnp.float32),
        mesh=mesh,
        compiler_params=pltpu.CompilerParams(use_tc_tiling_on_sc=False),
        scratch_types=[
            pltpu.VMEM((RB * LP,), jnp.int32),
            pltpu.VMEM((C0, DIM), jnp.float32),
            pltpu.VMEM((C1, DIM), jnp.float32),
            pltpu.VMEM((C0, DIM), jnp.float32),
            pltpu.VMEM((C1, DIM), jnp.float32),
            pltpu.VMEM((RB, DIM), jnp.float32),
            pltpu.SemaphoreType.DMA,
            pltpu.SemaphoreType.DMA,
            pltpu.SemaphoreType.DMA,
            pltpu.SemaphoreType.DMA,
        ],
    )(ids_tiled, table)


def _tc_body(ids_ref, s_ref, w_ref, b_ref, o_ref):
    cnt = jnp.sum((ids_ref[...] != 0).astype(jnp.float32), axis=1,
                  keepdims=True)
    pooled = s_ref[...] * (1.0 / jnp.maximum(cnt, 1.0))
    o_ref[...] = (
        jnp.dot(pooled, w_ref[...].T, preferred_element_type=jnp.float32)
        + b_ref[...]
    )


@jax.jit
def _tc_head(ids, sums, W, b):
    return pl.pallas_call(
        _tc_body,
        out_shape=jax.ShapeDtypeStruct((B, W.shape[0]), jnp.float32),
    )(ids, sums, W, b.reshape(1, -1))


def kernel(ids, table, W, b):
    ids32 = ids.astype(jnp.int32)
    ids_tiled = (
        jnp.pad(ids32, ((0, 0), (0, LP - L)))
        .reshape(B // 8, 8, 2, 128)
        .transpose(0, 2, 1, 3)
        .reshape(-1)
    )
    sums = _sc_pool(ids_tiled, table)
    return _tc_head(ids32, sums, W, b)
